# baseline (speedup 1.0000x reference)
import functools

import jax
import jax.numpy as jnp
from jax import lax
from jax.experimental import pallas as pl
from jax.experimental.pallas import tpu as pltpu
from jax.experimental.pallas import tpu_sc as plsc

_LANE = 128
_NC = 2
_NS = 16
_NW = _NC * _NS
_SUB = 128
_NBUF = 2
_UNROLL = 8


@functools.cache
def _make_gather(bsz, hist, emb):
    rows_per_w = bsz // _NW
    nblk = rows_per_w // _NBUF
    ib = _NBUF * hist
    mesh = plsc.VectorSubcoreMesh(core_axis_name="c", subcore_axis_name="s")

    @functools.partial(
        pl.kernel,
        mesh=mesh,
        out_type=jax.ShapeDtypeStruct((bsz, hist, emb), jnp.float32),
        compiler_params=pltpu.CompilerParams(use_tc_tiling_on_sc=True),
        scratch_types=[
            pltpu.VMEM((ib,), jnp.int32),
            pltpu.VMEM((_NBUF, hist, _LANE), jnp.float32),
            pltpu.VMEM((_NBUF, hist, emb), jnp.float32),
            pltpu.SemaphoreType.DMA,
            pltpu.SemaphoreType.DMA,
            pltpu.SemaphoreType.DMA,
        ],
    )
    def gather(ids_hbm, table_hbm, out_hbm, idx_v, rows_v, out_v, gsem,
               osem0, osem1):
        wid = lax.axis_index("s") * _NC + lax.axis_index("c")
        base_row = wid * rows_per_w
        osems = (osem0, osem1)
        ngroup = emb // 16
        nsub = (hist + _SUB - 1) // _SUB

        def body(blk, carry):
            bi = base_row + blk * _NBUF
            pltpu.sync_copy(ids_hbm.at[pl.ds(bi * hist, ib)], idx_v)
            for b in range(_NBUF):
                @pl.when(blk > 0)
                def _():
                    pltpu.make_async_copy(
                        out_v.at[b],
                        out_hbm.at[base_row],
                        osems[b],
                    ).wait()

                handles = []
                for j in range(nsub):
                    off = j * _SUB
                    sz = min(_SUB, hist - off)
                    handles.append(pltpu.async_copy(
                        table_hbm.at[idx_v.at[pl.ds(b * hist + off, sz)]],
                        rows_v.at[b].at[pl.ds(off, sz)],
                        gsem,
                    ))
                for h in handles:
                    h.wait()

                def compact(r, carry2):
                    for u in range(_UNROLL):
                        rr = r * _UNROLL + u
                        for j in range(ngroup):
                            out_v[b, rr, pl.ds(j * 16, 16)] = (
                                rows_v[b, rr, pl.ds(j * 16, 16)])
                    return carry2

                lax.fori_loop(0, hist // _UNROLL, compact, 0)
                pltpu.async_copy(
                    out_v.at[b],
                    out_hbm.at[bi + b],
                    osems[b],
                )
            return carry

        lax.fori_loop(0, nblk, body, 0)
        for b in range(_NBUF):
            pltpu.make_async_copy(
                out_v.at[b],
                out_hbm.at[base_row],
                osems[b],
            ).wait()

    return gather


def kernel(title_ids, title_embedding):
    b, h = title_ids.shape
    emb = title_embedding.shape[1]
    ids = title_ids.reshape(-1).astype(jnp.int32)
    table = jnp.pad(title_embedding, ((0, 0), (0, _LANE - emb)))
    return _make_gather(b, h, emb)(ids, table)
